# Initial kernel scaffold; baseline (speedup 1.0000x reference)
import jax
import jax.numpy as jnp
from jax.experimental import pallas as pl


def kernel(inputs, state, edge_index, Wg, bg, Wc, bc):
    raise NotImplementedError("write your pallas kernel here")


# speedup vs baseline: 2.7955x; 2.7955x over previous
import functools

import jax
import jax.numpy as jnp
from jax import lax
from jax.experimental import pallas as pl
from jax.experimental.pallas import tpu as pltpu
from jax.experimental.pallas import tpu_sc as plsc

N = 10000
E = 160000
B = 4
HID = 64
IN_DIM = 128
NP_ = 10240
NACC = NP_
NC = 2
NS = 16
CHUNK = 128
NCH = 40
EPT = NCH * CHUNK
EPAD = NC * NS * EPT
TN = 640
NT = NP_ // TN
F32 = jnp.float32


def _proj_g_body(i_ref, s_ref, wgi_ref, wgs_ref, wci_ref, zg_ref, zci_ref):
    x_i = i_ref[0]
    x_s = s_ref[0]
    zg = (jnp.dot(x_i, wgi_ref[0], preferred_element_type=F32)
          + jnp.dot(x_s, wgs_ref[0], preferred_element_type=F32))
    zg_ref[0, 0] = zg
    zci_ref[0, 0] = jnp.dot(x_i, wci_ref[0], preferred_element_type=F32)


def _proj_gate(iP, sP, WgI, WgS, WcI):
    return pl.pallas_call(
        _proj_g_body,
        grid=(B, 5, NT),
        in_specs=[
            pl.BlockSpec((1, TN, IN_DIM), lambda b, k, n: (b, n, 0)),
            pl.BlockSpec((1, TN, HID), lambda b, k, n: (b, n, 0)),
            pl.BlockSpec((1, IN_DIM, 128), lambda b, k, n: (k, 0, 0)),
            pl.BlockSpec((1, HID, 128), lambda b, k, n: (k, 0, 0)),
            pl.BlockSpec((1, IN_DIM, HID), lambda b, k, n: (k, 0, 0)),
        ],
        out_specs=[
            pl.BlockSpec((1, 1, TN, 128), lambda b, k, n: (b, k, n, 0)),
            pl.BlockSpec((1, 1, TN, HID), lambda b, k, n: (b, k, n, 0)),
        ],
        out_shape=[
            jax.ShapeDtypeStruct((B, 5, NP_, 128), F32),
            jax.ShapeDtypeStruct((B, 5, NP_, HID), F32),
        ],
    )(iP, sP, WgI, WgS, WcI)


def _dinv_col(deg_blk):
    d = deg_blk[:, :1]
    return jnp.where(d > 0.0, 1.0 / d, 0.0)


def _comb1_body(z_ref, t_ref, deg_ref, out_ref):
    dinv = _dinv_col(deg_ref[0])
    t = t_ref[0, 0, 0] + t_ref[0, 0, 1]
    out_ref[0, 0] = z_ref[0, 0] + 2.0 * dinv * t


def _comb1(Z, t, deg64, p0):
    return pl.pallas_call(
        _comb1_body,
        grid=(p0, 2, NT),
        in_specs=[
            pl.BlockSpec((1, 1, TN, 128), lambda pb, m, n: (pb, 1 + 2 * m, n, 0)),
            pl.BlockSpec((1, 1, 2, TN, 128), lambda pb, m, n: (pb, m, 0, n, 0)),
            pl.BlockSpec((1, TN, 128), lambda pb, m, n: (m, n, 0)),
        ],
        out_specs=pl.BlockSpec((1, 1, TN, 128), lambda pb, m, n: (pb, m, n, 0)),
        out_shape=jax.ShapeDtypeStruct((p0, 2, NP_, 128), F32),
    )(Z, t, deg64)


def _comb2g_body(zg_ref, w_ref, deg_ref, s_ref, bg_ref, u_ref, rs_ref):
    dinv_a = _dinv_col(deg_ref[0])
    dinv_t = _dinv_col(deg_ref[1])
    zg = zg_ref[0]
    w = w_ref[0]
    wa = w[0, 0] + w[0, 1]
    wt = w[1, 0] + w[1, 1]
    g = zg[0] - zg[2] - zg[4] + dinv_a * wa + dinv_t * wt + bg_ref[...]
    v = jax.nn.sigmoid(g)
    u_ref[0] = v[:, HID:]
    rs_ref[0] = v[:, :HID] * s_ref[0]


def _comb2g(ZG, w, deg64, sP, bg):
    return pl.pallas_call(
        _comb2g_body,
        grid=(B, NT),
        in_specs=[
            pl.BlockSpec((1, 5, TN, 128), lambda b, n: (b, 0, n, 0)),
            pl.BlockSpec((1, 2, 2, TN, 128), lambda b, n: (b, 0, 0, n, 0)),
            pl.BlockSpec((2, TN, 128), lambda b, n: (0, n, 0)),
            pl.BlockSpec((1, TN, HID), lambda b, n: (b, n, 0)),
            pl.BlockSpec((128,), lambda b, n: (0,)),
        ],
        out_specs=[
            pl.BlockSpec((1, TN, HID), lambda b, n: (b, n, 0)),
            pl.BlockSpec((1, TN, HID), lambda b, n: (b, n, 0)),
        ],
        out_shape=[
            jax.ShapeDtypeStruct((B, NP_, HID), F32),
            jax.ShapeDtypeStruct((B, NP_, HID), F32),
        ],
    )(ZG, w, deg64, sP, bg)


def _projc_body(zci_ref, rs_ref, wcs_ref, zc_ref):
    w = wcs_ref[0]
    a0 = zci_ref[0, 0] + jnp.dot(rs_ref[0], w, preferred_element_type=F32)
    a1 = zci_ref[1, 0] + jnp.dot(rs_ref[1], w, preferred_element_type=F32)
    zc_ref[0, 0] = jnp.concatenate([a0, a1], axis=1)


def _projc(ZCI, rs, WcS):
    return pl.pallas_call(
        _projc_body,
        grid=(2, 5, NT),
        in_specs=[
            pl.BlockSpec((2, 1, TN, HID), lambda p, k, n: (p, k, n, 0)),
            pl.BlockSpec((2, TN, HID), lambda p, k, n: (p, n, 0)),
            pl.BlockSpec((1, HID, HID), lambda p, k, n: (k, 0, 0)),
        ],
        out_specs=pl.BlockSpec((1, 1, TN, 128), lambda p, k, n: (p, k, n, 0)),
        out_shape=jax.ShapeDtypeStruct((2, 5, NP_, 128), F32),
    )(ZCI, rs, WcS)


def _final_body(zc_ref, wc_ref, deg_ref, u_ref, s_ref, bc2_ref, ns_ref):
    dinv_a = _dinv_col(deg_ref[0])
    dinv_t = _dinv_col(deg_ref[1])
    zc = zc_ref[0]
    wc = wc_ref[0]
    wa = wc[0, 0] + wc[0, 1]
    wt = wc[1, 0] + wc[1, 1]
    cpre = zc[0] - zc[2] - zc[4] + dinv_a * wa + dinv_t * wt + bc2_ref[...]
    cc = jnp.tanh(cpre)
    u = u_ref[...]
    s = s_ref[...]
    n0 = u[0] * s[0] + (1.0 - u[0]) * cc[:, :HID]
    n1 = u[1] * s[1] + (1.0 - u[1]) * cc[:, HID:]
    ns_ref[...] = jnp.stack([n0, n1], axis=0)


def _final(ZC, wc, deg64, u, sP, bc2):
    return pl.pallas_call(
        _final_body,
        grid=(2, NT),
        in_specs=[
            pl.BlockSpec((1, 5, TN, 128), lambda p, n: (p, 0, n, 0)),
            pl.BlockSpec((1, 2, 2, TN, 128), lambda p, n: (p, 0, 0, n, 0)),
            pl.BlockSpec((2, TN, 128), lambda p, n: (0, n, 0)),
            pl.BlockSpec((2, TN, HID), lambda p, n: (p, n, 0)),
            pl.BlockSpec((2, TN, HID), lambda p, n: (p, n, 0)),
            pl.BlockSpec((128,), lambda p, n: (0,)),
        ],
        out_specs=pl.BlockSpec((2, TN, HID), lambda p, n: (p, n, 0)),
        out_shape=jax.ShapeDtypeStruct((B, NP_, HID), F32),
    )(ZC, wc, deg64, u, sP, bc2)


_ZSLICES = (NACC // NS) // CHUNK


def _deg_kernel_body(rows4, cols4, ones_blk, zeros_blk, out,
                     acc, rbuf, cbuf, dstbuf, onesbuf):
    c = lax.axis_index("c")
    s = lax.axis_index("s")
    pltpu.sync_copy(ones_blk, onesbuf)
    for q in range(_ZSLICES):
        pltpu.sync_copy(zeros_blk, acc.at[pl.ds(s * 640 + q * CHUNK, CHUNK)])
    plsc.subcore_barrier()

    for h in range(NC):
        pltpu.sync_copy(rows4.at[h].at[s], rbuf)
        pltpu.sync_copy(cols4.at[h].at[s], cbuf)

        @pl.loop(0, NCH)
        def _chunk(ch):
            for j in range(CHUNK // 16):
                rv = rbuf[ch, pl.ds(j * 16, 16)]
                cv = cbuf[ch, pl.ds(j * 16, 16)]
                dstbuf[pl.ds(j * 16, 16)] = rv + (cv - rv) * c
            pltpu.sync_copy(onesbuf, acc.at[dstbuf], add=True)

    plsc.subcore_barrier()
    pltpu.sync_copy(acc.at[pl.ds(s * 640, 640)],
                    out.at[c].at[pl.ds(s * 640, 640)])


@functools.cache
def _deg_call():
    mesh = plsc.VectorSubcoreMesh(core_axis_name="c", subcore_axis_name="s",
                                  num_cores=NC, num_subcores=NS)
    return pl.kernel(
        _deg_kernel_body,
        out_type=jax.ShapeDtypeStruct((NC, NACC, 128), F32),
        mesh=mesh,
        scratch_types=[
            pltpu.VMEM_SHARED((NACC, 128), F32),
            pltpu.VMEM((NCH, CHUNK), jnp.int32),
            pltpu.VMEM((NCH, CHUNK), jnp.int32),
            pltpu.VMEM((CHUNK,), jnp.int32),
            pltpu.VMEM((CHUNK, 128), F32),
        ],
    )


@functools.cache
def _make_spmm(passes):
    num_p = len(passes)
    mesh = plsc.VectorSubcoreMesh(core_axis_name="c", subcore_axis_name="s",
                                  num_cores=NC, num_subcores=NS)

    def body(tables, rows4, cols4, zeros_blk, out,
             acc, srcbuf, dstbuf, rowbuf, sem0, sem1):
        c = lax.axis_index("c")
        s = lax.axis_index("s")
        sems = (sem0, sem1)

        for p, (off, m) in enumerate(passes):
            addc = NP_ * off
            srcarr = cols4 if m == 0 else rows4
            dstarr = rows4 if m == 0 else cols4
            for q in range(_ZSLICES):
                pltpu.sync_copy(zeros_blk,
                                acc.at[pl.ds(s * 640 + q * CHUNK, CHUNK)])
            pltpu.sync_copy(srcarr.at[c].at[s], srcbuf)
            pltpu.sync_copy(dstarr.at[c].at[s], dstbuf)

            @pl.loop(0, NCH)
            def _prep(ch):
                for j in range(CHUNK // 16):
                    v = srcbuf[ch, pl.ds(j * 16, 16)]
                    srcbuf[ch, pl.ds(j * 16, 16)] = v + addc

            plsc.subcore_barrier()

            def _gather(ch, slot):
                return pltpu.async_copy(
                    tables.at[srcbuf.at[ch]], rowbuf.at[slot], sems[slot])

            def _scat(ch, slot):
                pltpu.sync_copy(rowbuf.at[slot], acc.at[dstbuf.at[ch]],
                                add=True)

            _gather(0, 0)

            @pl.loop(0, NCH, step=2)
            def _main(ch):
                pltpu.make_async_copy(
                    tables.at[srcbuf.at[ch]], rowbuf.at[0], sems[0]).wait()
                _gather(ch + 1, 1)
                _scat(ch, 0)
                pltpu.make_async_copy(
                    tables.at[srcbuf.at[ch + 1]], rowbuf.at[1], sems[1]).wait()

                @pl.when(ch + 2 < NCH)
                def _():
                    _gather(ch + 2, 0)

                _scat(ch + 1, 1)

            plsc.subcore_barrier()
            pltpu.sync_copy(acc.at[pl.ds(s * 640, 640)],
                            out.at[p].at[c].at[pl.ds(s * 640, 640)])

    return pl.kernel(
        body,
        out_type=jax.ShapeDtypeStruct((num_p, NC, NP_, 128), F32),
        mesh=mesh,
        scratch_types=[
            pltpu.VMEM_SHARED((NACC, 128), F32),
            pltpu.VMEM((NCH, CHUNK), jnp.int32),
            pltpu.VMEM((NCH, CHUNK), jnp.int32),
            pltpu.VMEM((2, CHUNK, 128), F32),
            pltpu.SemaphoreType.DMA,
            pltpu.SemaphoreType.DMA,
        ],
    )


_PASSES_G1 = tuple((b * 5 + 2 + 2 * m, m) for b in range(B) for m in range(2))
_PASSES_G2 = tuple((b * 2 + m, m) for b in range(B) for m in range(2))
_PASSES_C1 = tuple((p * 5 + 2 + 2 * m, m) for p in range(2) for m in range(2))
_PASSES_C2 = tuple((p * 2 + m, m) for p in range(2) for m in range(2))


def kernel(inputs, state, edge_index, Wg, bg, Wc, bc):
    i = inputs.reshape(B, N, IN_DIM)
    s = state.reshape(B, N, HID)
    iP = jnp.pad(i, ((0, 0), (0, NP_ - N), (0, 0)))
    sP = jnp.pad(s, ((0, 0), (0, NP_ - N), (0, 0)))
    pad = jnp.full((EPAD - E,), N, jnp.int32)
    rows4 = jnp.concatenate([edge_index[0], pad]).reshape(NC, NS, NCH, CHUNK)
    cols4 = jnp.concatenate([edge_index[1], pad]).reshape(NC, NS, NCH, CHUNK)

    WgR = Wg.reshape(192, 5, 128).transpose(1, 0, 2)
    WgI, WgS = WgR[:, :IN_DIM], WgR[:, IN_DIM:]
    WcR = Wc.reshape(192, 5, HID).transpose(1, 0, 2)
    WcI, WcS = WcR[:, :IN_DIM], WcR[:, IN_DIM:]
    bc2 = jnp.concatenate([bc, bc])

    zeros_blk = jnp.zeros((CHUNK, 128), F32)
    ones_blk = jnp.ones((CHUNK, 128), F32)

    deg64 = _deg_call()(rows4, cols4, ones_blk, zeros_blk)
    ZG, ZCI = _proj_gate(iP, sP, WgI, WgS, WcI)

    t = _make_spmm(_PASSES_G1)(ZG.reshape(-1, 128), rows4, cols4, zeros_blk)
    t = t.reshape(B, 2, NC, NP_, 128)
    CG = _comb1(ZG, t, deg64, B)
    w = _make_spmm(_PASSES_G2)(CG.reshape(-1, 128), rows4, cols4, zeros_blk)
    w = w.reshape(B, 2, NC, NP_, 128)
    u, rs = _comb2g(ZG, w, deg64, sP, bg)

    ZC = _projc(ZCI, rs, WcS)
    tc = _make_spmm(_PASSES_C1)(ZC.reshape(-1, 128), rows4, cols4, zeros_blk)
    tc = tc.reshape(2, 2, NC, NP_, 128)
    CC = _comb1(ZC, tc, deg64, 2)
    wc = _make_spmm(_PASSES_C2)(CC.reshape(-1, 128), rows4, cols4, zeros_blk)
    wc = wc.reshape(2, 2, NC, NP_, 128)

    ns = _final(ZC, wc, deg64, u, sP, bc2)
    ns = ns[:, :N, :].reshape(B, N * HID)
    return (ns, ns)
